# baseline (speedup 1.0000x reference)
import functools

import jax
import jax.numpy as jnp
from jax import lax
from jax.experimental import pallas as pl
from jax.experimental.pallas import tpu as pltpu
from jax.experimental.pallas import tpu_sc as plsc

NC = 2
NS = 16
NW = NC * NS
L = 16

D = 64
TCB = 512
CHUNK = 64


@functools.lru_cache(maxsize=None)
def _make_tc_pack(n):
    grid = (n + TCB - 1) // TCB
    half = TCB // 2
    last = n // half

    @functools.partial(
        pl.pallas_call,
        grid=(grid,),
        in_specs=[
            pl.BlockSpec((D, half), lambda i: (0, jnp.minimum(2 * i, last))),
            pl.BlockSpec((D, half),
                         lambda i: (0, jnp.minimum(2 * i + 1, last))),
        ],
        out_specs=pl.BlockSpec((half, 2 * D), lambda i: (i, 0)),
        out_shape=jax.ShapeDtypeStruct((grid * half, 2 * D), jnp.float32),
    )
    def pack(a_ref, b_ref, o_ref):
        o_ref[:, 0:D] = a_ref[...].T
        o_ref[:, D:2 * D] = b_ref[...].T

    return pack


@functools.lru_cache(maxsize=None)
def _make_sc_scorer(total):
    assert total % (NW * CHUNK) == 0
    per_w = total // NW
    n_chunks = per_w // CHUNK
    mesh = plsc.VectorSubcoreMesh(core_axis_name="c", subcore_axis_name="s")

    @functools.partial(
        pl.kernel,
        mesh=mesh,
        out_type=jax.ShapeDtypeStruct((total,), jnp.float32),
        compiler_params=pltpu.CompilerParams(needs_layout_passes=False),
        scratch_types=[
            pltpu.VMEM((CHUNK,), jnp.int32),
            pltpu.VMEM((CHUNK,), jnp.int32),
            pltpu.VMEM((CHUNK,), jnp.int32),
            pltpu.VMEM((CHUNK,), jnp.int32),
            pltpu.VMEM((CHUNK,), jnp.int32),
            pltpu.VMEM((CHUNK,), jnp.int32),
            pltpu.VMEM((CHUNK, 2 * D), jnp.float32),
            pltpu.VMEM((CHUNK, 2 * D), jnp.float32),
            pltpu.VMEM((CHUNK, 2 * D), jnp.float32),
            pltpu.VMEM((CHUNK, 2 * D), jnp.float32),
            pltpu.VMEM((CHUNK, 2 * D), jnp.float32),
            pltpu.VMEM((CHUNK, 2 * D), jnp.float32),
            pltpu.VMEM((per_w,), jnp.float32),
            pltpu.SemaphoreType.DMA,
        ],
    )
    def scorer(hp_hbm, tp_hbm, rp_hbm, hc_hbm, tc_hbm, rc_hbm,
               eh_hbm, et_hbm, relh_hbm, relt_hbm,
               out_hbm, hi, ti, ri, hdiv, tdiv, rdiv,
               hh, ht, th, tt, rh, rt, sv, sem):
        wid = lax.axis_index("s") * NC + lax.axis_index("c")
        base = wid * per_w
        lanes = lax.iota(jnp.int32, L)

        def chunk_body(c, carry):
            off = base + c * CHUNK
            pltpu.sync_copy(hc_hbm.at[pl.ds(off, CHUNK)], hi)
            pltpu.sync_copy(tc_hbm.at[pl.ds(off, CHUNK)], ti)
            pltpu.sync_copy(rc_hbm.at[pl.ds(off, CHUNK)], ri)
            pltpu.sync_copy(hp_hbm.at[pl.ds(off, CHUNK)], hdiv)
            pltpu.sync_copy(tp_hbm.at[pl.ds(off, CHUNK)], tdiv)
            pltpu.sync_copy(rp_hbm.at[pl.ds(off, CHUNK)], rdiv)
            copies = [
                pltpu.async_copy(eh_hbm.at[hdiv], hh, sem),
                pltpu.async_copy(et_hbm.at[hdiv], ht, sem),
                pltpu.async_copy(eh_hbm.at[tdiv], th, sem),
                pltpu.async_copy(et_hbm.at[tdiv], tt, sem),
                pltpu.async_copy(relh_hbm.at[rdiv], rh, sem),
                pltpu.async_copy(relt_hbm.at[rdiv], rt, sem),
            ]
            for cp in copies:
                cp.wait()

            def group_body(g, carry2):
                i0 = g * L
                s = pl.ds(i0, L)
                rows = lanes + i0
                hcol = hi[s]
                tcol = ti[s]
                rcol = ri[s]
                acc = jnp.zeros((L,), jnp.float32)
                for d in range(D):
                    hhd = plsc.load_gather(hh, [rows, hcol + d])
                    htd = plsc.load_gather(ht, [rows, hcol + d])
                    thd = plsc.load_gather(th, [rows, tcol + d])
                    ttd = plsc.load_gather(tt, [rows, tcol + d])
                    rhd = plsc.load_gather(rh, [rows, rcol + d])
                    rtd = plsc.load_gather(rt, [rows, rcol + d])
                    acc = acc + (hhd * rhd * ttd + thd * rtd * htd)
                sv[pl.ds(c * CHUNK + i0, L)] = 0.5 * acc
                return carry2

            lax.fori_loop(0, CHUNK // L, group_body, 0)
            return carry

        lax.fori_loop(0, n_chunks, chunk_body, 0)
        pltpu.sync_copy(sv, out_hbm.at[pl.ds(base, per_w)])

    return scorer


def kernel(pos_h, pos_r, pos_t, neg_h, neg_r, neg_t,
           entity_head, entity_tail, relation_head, relation_tail):
    b = pos_h.shape[0]
    h = jnp.concatenate([pos_h, neg_h])
    t = jnp.concatenate([pos_t, neg_t])
    r = jnp.concatenate([pos_r, neg_r])
    hp = ((h >> 9) << 8) + (h & 255)
    tp = ((t >> 9) << 8) + (t & 255)
    rp = ((r >> 9) << 8) + (r & 255)
    hc = ((h >> 8) & 1) * D
    tc = ((t >> 8) & 1) * D
    rc = ((r >> 8) & 1) * D
    epack = _make_tc_pack(entity_head.shape[0])
    rpack = _make_tc_pack(relation_head.shape[0])
    ehp = epack(entity_head.T, entity_head.T)
    etp = epack(entity_tail.T, entity_tail.T)
    rhp = rpack(relation_head.T, relation_head.T)
    rtp = rpack(relation_tail.T, relation_tail.T)
    scorer = _make_sc_scorer(2 * b)
    out = scorer(hp, tp, rp, hc, tc, rc, ehp, etp, rhp, rtp)
    return out[:b], out[b:]


# speedup vs baseline: 1.5968x; 1.5968x over previous
import functools

import jax
import jax.numpy as jnp
from jax import lax
from jax.experimental import pallas as pl
from jax.experimental.pallas import tpu as pltpu
from jax.experimental.pallas import tpu_sc as plsc

NC = 2
NS = 16
NW = NC * NS
L = 16

D = 64
TCB = 1024
HALF = TCB // 2
CHUNK = 64


@functools.lru_cache(maxsize=None)
def _make_tc_pack(n):
    grid = (n + TCB - 1) // TCB
    last = n // HALF

    @functools.partial(
        pl.pallas_call,
        grid=(grid,),
        in_specs=[
            pl.BlockSpec((D, HALF), lambda i: (0, jnp.minimum(2 * i, last))),
            pl.BlockSpec((D, HALF),
                         lambda i: (0, jnp.minimum(2 * i + 1, last))),
        ],
        out_specs=pl.BlockSpec((HALF, 2 * D), lambda i: (i, 0)),
        out_shape=jax.ShapeDtypeStruct((grid * HALF, 2 * D), jnp.float32),
    )
    def pack(a_ref, b_ref, o_ref):
        eye = (lax.iota(jnp.int32, D)[:, None]
               == lax.iota(jnp.int32, D)[None, :]).astype(jnp.float32)
        dn = (((0,), (0,)), ((), ()))
        o_ref[:, 0:D] = lax.dot_general(
            a_ref[...], eye, dn, preferred_element_type=jnp.float32)
        o_ref[:, D:2 * D] = lax.dot_general(
            b_ref[...], eye, dn, preferred_element_type=jnp.float32)

    return pack


@functools.lru_cache(maxsize=None)
def _make_sc_scorer(total):
    assert total % (NW * 2 * CHUNK) == 0
    per_w = total // NW
    n_chunks = per_w // CHUNK
    mesh = plsc.VectorSubcoreMesh(core_axis_name="c", subcore_axis_name="s")

    rowbuf = pltpu.VMEM((CHUNK, 2 * D), jnp.float32)
    idxbuf = pltpu.VMEM((per_w,), jnp.int32)

    @functools.partial(
        pl.kernel,
        mesh=mesh,
        out_type=jax.ShapeDtypeStruct((total,), jnp.float32),
        compiler_params=pltpu.CompilerParams(needs_layout_passes=False),
        scratch_types=[
            idxbuf, idxbuf, idxbuf,
            idxbuf, idxbuf, idxbuf,
            [rowbuf] * 6,
            [rowbuf] * 6,
            pltpu.VMEM((per_w,), jnp.float32),
            pltpu.SemaphoreType.DMA,
            pltpu.SemaphoreType.DMA,
        ],
    )
    def scorer(hp_hbm, tp_hbm, rp_hbm, hc_hbm, tc_hbm, rc_hbm,
               eh_hbm, et_hbm, relh_hbm, relt_hbm,
               out_hbm, hpv, tpv, rpv, hcv, tcv, rcv,
               bufs0, bufs1, sv, sem0, sem1):
        wid = lax.axis_index("s") * NC + lax.axis_index("c")
        base = wid * per_w
        lanes = lax.iota(jnp.int32, L)
        bufsets = (bufs0, bufs1)
        sems = (sem0, sem1)

        pltpu.sync_copy(hp_hbm.at[pl.ds(base, per_w)], hpv)
        pltpu.sync_copy(tp_hbm.at[pl.ds(base, per_w)], tpv)
        pltpu.sync_copy(rp_hbm.at[pl.ds(base, per_w)], rpv)
        pltpu.sync_copy(hc_hbm.at[pl.ds(base, per_w)], hcv)
        pltpu.sync_copy(tc_hbm.at[pl.ds(base, per_w)], tcv)
        pltpu.sync_copy(rc_hbm.at[pl.ds(base, per_w)], rcv)

        def issue(c, which):
            off = pl.ds(c * CHUNK, CHUNK)
            bufs, sem = bufsets[which], sems[which]
            return [
                pltpu.async_copy(eh_hbm.at[hpv.at[off]], bufs[0], sem),
                pltpu.async_copy(et_hbm.at[hpv.at[off]], bufs[1], sem),
                pltpu.async_copy(eh_hbm.at[tpv.at[off]], bufs[2], sem),
                pltpu.async_copy(et_hbm.at[tpv.at[off]], bufs[3], sem),
                pltpu.async_copy(relh_hbm.at[rpv.at[off]], bufs[4], sem),
                pltpu.async_copy(relt_hbm.at[rpv.at[off]], bufs[5], sem),
            ]

        def compute(c, which):
            hh, ht, th, tt, rh, rt = bufsets[which]

            def group_body(g, carry):
                i0 = g * L
                s = pl.ds(c * CHUNK + i0, L)
                rows = lanes + i0
                hcol = hcv[s]
                tcol = tcv[s]
                rcol = rcv[s]
                acc = jnp.zeros((L,), jnp.float32)
                for d in range(D):
                    hhd = plsc.load_gather(hh, [rows, hcol + d])
                    htd = plsc.load_gather(ht, [rows, hcol + d])
                    thd = plsc.load_gather(th, [rows, tcol + d])
                    ttd = plsc.load_gather(tt, [rows, tcol + d])
                    rhd = plsc.load_gather(rh, [rows, rcol + d])
                    rtd = plsc.load_gather(rt, [rows, rcol + d])
                    acc = acc + (hhd * rhd * ttd + thd * rtd * htd)
                sv[pl.ds(c * CHUNK + i0, L)] = 0.5 * acc
                return carry

            lax.fori_loop(0, CHUNK // L, group_body, 0)

        def wait_chunk(c, which):
            off = pl.ds(c * CHUNK, CHUNK)
            bufs, sem = bufsets[which], sems[which]
            pltpu.make_async_copy(eh_hbm.at[hpv.at[off]], bufs[0], sem).wait()
            pltpu.make_async_copy(et_hbm.at[hpv.at[off]], bufs[1], sem).wait()
            pltpu.make_async_copy(eh_hbm.at[tpv.at[off]], bufs[2], sem).wait()
            pltpu.make_async_copy(et_hbm.at[tpv.at[off]], bufs[3], sem).wait()
            pltpu.make_async_copy(relh_hbm.at[rpv.at[off]], bufs[4],
                                  sem).wait()
            pltpu.make_async_copy(relt_hbm.at[rpv.at[off]], bufs[5],
                                  sem).wait()

        issue(0, 0)

        def pair(k, carry):
            c0 = 2 * k
            issue(c0 + 1, 1)
            wait_chunk(c0, 0)
            compute(c0, 0)
            nxt = jnp.minimum(c0 + 2, n_chunks - 2)
            issue(nxt, 0)
            wait_chunk(c0 + 1, 1)
            compute(c0 + 1, 1)
            return carry

        lax.fori_loop(0, n_chunks // 2, pair, 0)
        wait_chunk(n_chunks - 2, 0)
        pltpu.sync_copy(sv, out_hbm.at[pl.ds(base, per_w)])

    return scorer


def kernel(pos_h, pos_r, pos_t, neg_h, neg_r, neg_t,
           entity_head, entity_tail, relation_head, relation_tail):
    b = pos_h.shape[0]
    h = jnp.concatenate([pos_h, neg_h])
    t = jnp.concatenate([pos_t, neg_t])
    r = jnp.concatenate([pos_r, neg_r])
    sh = HALF.bit_length() - 1
    hp = ((h >> (sh + 1)) << sh) + (h & (HALF - 1))
    tp = ((t >> (sh + 1)) << sh) + (t & (HALF - 1))
    rp = ((r >> (sh + 1)) << sh) + (r & (HALF - 1))
    hc = ((h >> sh) & 1) * D
    tc = ((t >> sh) & 1) * D
    rc = ((r >> sh) & 1) * D
    epack = _make_tc_pack(entity_head.shape[0])
    rpack = _make_tc_pack(relation_head.shape[0])
    ehp = epack(entity_head.T, entity_head.T)
    etp = epack(entity_tail.T, entity_tail.T)
    rhp = rpack(relation_head.T, relation_head.T)
    rtp = rpack(relation_tail.T, relation_tail.T)
    scorer = _make_sc_scorer(2 * b)
    out = scorer(hp, tp, rp, hc, tc, rc, ehp, etp, rhp, rtp)
    return out[:b], out[b:]


# speedup vs baseline: 3.0741x; 1.9252x over previous
import functools

import jax
import jax.numpy as jnp
from jax import lax
from jax.experimental import pallas as pl
from jax.experimental.pallas import tpu as pltpu
from jax.experimental.pallas import tpu_sc as plsc

NC = 2
NS = 16
NW = NC * NS
L = 16

D = 64
TCB = 4096
HALF = TCB // 2
CHUNK = 64


@functools.lru_cache(maxsize=None)
def _make_tc_pack(n):
    grid = (n + TCB - 1) // TCB
    last = n // HALF

    @functools.partial(
        pl.pallas_call,
        grid=(grid,),
        in_specs=[
            pl.BlockSpec((D, HALF), lambda i: (0, jnp.minimum(2 * i, last))),
            pl.BlockSpec((D, HALF),
                         lambda i: (0, jnp.minimum(2 * i + 1, last))),
        ],
        out_specs=pl.BlockSpec((HALF, 2 * D), lambda i: (i, 0)),
        out_shape=jax.ShapeDtypeStruct((grid * HALF, 2 * D), jnp.float32),
        compiler_params=pltpu.CompilerParams(
            dimension_semantics=("parallel",)),
    )
    def pack(a_ref, b_ref, o_ref):
        o_ref[:, 0:D] = a_ref[...].T
        o_ref[:, D:2 * D] = b_ref[...].T

    return pack


@functools.lru_cache(maxsize=None)
def _make_sc_scorer(total):
    assert total % (NW * 2 * CHUNK) == 0
    per_w = total // NW
    n_chunks = per_w // CHUNK
    mesh = plsc.VectorSubcoreMesh(core_axis_name="c", subcore_axis_name="s")

    rowbuf = pltpu.VMEM((CHUNK, 2 * D), jnp.float32)
    idxbuf = pltpu.VMEM((per_w,), jnp.int32)

    @functools.partial(
        pl.kernel,
        mesh=mesh,
        out_type=jax.ShapeDtypeStruct((total,), jnp.float32),
        compiler_params=pltpu.CompilerParams(needs_layout_passes=False),
        scratch_types=[
            idxbuf, idxbuf, idxbuf,
            idxbuf, idxbuf, idxbuf,
            [rowbuf] * 6,
            [rowbuf] * 6,
            pltpu.VMEM((per_w,), jnp.float32),
            pltpu.SemaphoreType.DMA,
            pltpu.SemaphoreType.DMA,
        ],
    )
    def scorer(hp_hbm, tp_hbm, rp_hbm, hc_hbm, tc_hbm, rc_hbm,
               eh_hbm, et_hbm, relh_hbm, relt_hbm,
               out_hbm, hpv, tpv, rpv, hcv, tcv, rcv,
               bufs0, bufs1, sv, sem0, sem1):
        wid = lax.axis_index("s") * NC + lax.axis_index("c")
        base = wid * per_w
        lanes = lax.iota(jnp.int32, L)
        bufsets = (bufs0, bufs1)
        sems = (sem0, sem1)

        pltpu.sync_copy(hp_hbm.at[pl.ds(base, per_w)], hpv)
        pltpu.sync_copy(tp_hbm.at[pl.ds(base, per_w)], tpv)
        pltpu.sync_copy(rp_hbm.at[pl.ds(base, per_w)], rpv)
        pltpu.sync_copy(hc_hbm.at[pl.ds(base, per_w)], hcv)
        pltpu.sync_copy(tc_hbm.at[pl.ds(base, per_w)], tcv)
        pltpu.sync_copy(rc_hbm.at[pl.ds(base, per_w)], rcv)

        def issue(c, which):
            off = pl.ds(c * CHUNK, CHUNK)
            bufs, sem = bufsets[which], sems[which]
            return [
                pltpu.async_copy(eh_hbm.at[hpv.at[off]], bufs[0], sem),
                pltpu.async_copy(et_hbm.at[hpv.at[off]], bufs[1], sem),
                pltpu.async_copy(eh_hbm.at[tpv.at[off]], bufs[2], sem),
                pltpu.async_copy(et_hbm.at[tpv.at[off]], bufs[3], sem),
                pltpu.async_copy(relh_hbm.at[rpv.at[off]], bufs[4], sem),
                pltpu.async_copy(relt_hbm.at[rpv.at[off]], bufs[5], sem),
            ]

        def compute(c, which):
            hh, ht, th, tt, rh, rt = bufsets[which]

            def group_body(g, carry):
                i0 = g * L
                s = pl.ds(c * CHUNK + i0, L)
                rows = lanes + i0
                hcol = hcv[s]
                tcol = tcv[s]
                rcol = rcv[s]
                acc = jnp.zeros((L,), jnp.float32)
                for d in range(D):
                    hhd = plsc.load_gather(hh, [rows, hcol + d])
                    htd = plsc.load_gather(ht, [rows, hcol + d])
                    thd = plsc.load_gather(th, [rows, tcol + d])
                    ttd = plsc.load_gather(tt, [rows, tcol + d])
                    rhd = plsc.load_gather(rh, [rows, rcol + d])
                    rtd = plsc.load_gather(rt, [rows, rcol + d])
                    acc = acc + (hhd * rhd * ttd + thd * rtd * htd)
                sv[pl.ds(c * CHUNK + i0, L)] = 0.5 * acc
                return carry

            lax.fori_loop(0, CHUNK // L, group_body, 0)

        def wait_chunk(c, which):
            off = pl.ds(c * CHUNK, CHUNK)
            bufs, sem = bufsets[which], sems[which]
            pltpu.make_async_copy(eh_hbm.at[hpv.at[off]], bufs[0], sem).wait()
            pltpu.make_async_copy(et_hbm.at[hpv.at[off]], bufs[1], sem).wait()
            pltpu.make_async_copy(eh_hbm.at[tpv.at[off]], bufs[2], sem).wait()
            pltpu.make_async_copy(et_hbm.at[tpv.at[off]], bufs[3], sem).wait()
            pltpu.make_async_copy(relh_hbm.at[rpv.at[off]], bufs[4],
                                  sem).wait()
            pltpu.make_async_copy(relt_hbm.at[rpv.at[off]], bufs[5],
                                  sem).wait()

        issue(0, 0)

        def pair(k, carry):
            c0 = 2 * k
            issue(c0 + 1, 1)
            wait_chunk(c0, 0)
            compute(c0, 0)
            nxt = jnp.minimum(c0 + 2, n_chunks - 2)
            issue(nxt, 0)
            wait_chunk(c0 + 1, 1)
            compute(c0 + 1, 1)
            return carry

        lax.fori_loop(0, n_chunks // 2, pair, 0)
        wait_chunk(n_chunks - 2, 0)
        pltpu.sync_copy(sv, out_hbm.at[pl.ds(base, per_w)])

    return scorer


def kernel(pos_h, pos_r, pos_t, neg_h, neg_r, neg_t,
           entity_head, entity_tail, relation_head, relation_tail):
    b = pos_h.shape[0]
    h = jnp.concatenate([pos_h, neg_h])
    t = jnp.concatenate([pos_t, neg_t])
    r = jnp.concatenate([pos_r, neg_r])
    sh = HALF.bit_length() - 1
    hp = ((h >> (sh + 1)) << sh) + (h & (HALF - 1))
    tp = ((t >> (sh + 1)) << sh) + (t & (HALF - 1))
    rp = ((r >> (sh + 1)) << sh) + (r & (HALF - 1))
    hc = ((h >> sh) & 1) * D
    tc = ((t >> sh) & 1) * D
    rc = ((r >> sh) & 1) * D
    epack = _make_tc_pack(entity_head.shape[0])
    rpack = _make_tc_pack(relation_head.shape[0])
    ehp = epack(entity_head.T, entity_head.T)
    etp = epack(entity_tail.T, entity_tail.T)
    rhp = rpack(relation_head.T, relation_head.T)
    rtp = rpack(relation_tail.T, relation_tail.T)
    scorer = _make_sc_scorer(2 * b)
    out = scorer(hp, tp, rp, hc, tc, rc, ehp, etp, rhp, rtp)
    return out[:b], out[b:]


# speedup vs baseline: 3.9325x; 1.2792x over previous
import functools

import jax
import jax.numpy as jnp
from jax import lax
from jax.experimental import pallas as pl
from jax.experimental.pallas import tpu as pltpu
from jax.experimental.pallas import tpu_sc as plsc

NC = 2
NS = 16
NW = NC * NS
L = 16

D = 64
TCB = 16384
HALF = TCB // 2
CHUNK = 64


@functools.lru_cache(maxsize=None)
def _make_tc_pack(n):
    grid = (n + TCB - 1) // TCB
    last = n // HALF

    @functools.partial(
        pl.pallas_call,
        grid=(grid,),
        in_specs=[
            pl.BlockSpec((D, HALF), lambda i: (0, jnp.minimum(2 * i, last))),
            pl.BlockSpec((D, HALF),
                         lambda i: (0, jnp.minimum(2 * i + 1, last))),
        ],
        out_specs=pl.BlockSpec((HALF, 2 * D), lambda i: (i, 0)),
        out_shape=jax.ShapeDtypeStruct((grid * HALF, 2 * D), jnp.float32),
        compiler_params=pltpu.CompilerParams(
            dimension_semantics=("parallel",)),
    )
    def pack(a_ref, b_ref, o_ref):
        o_ref[:, 0:D] = a_ref[...].T
        o_ref[:, D:2 * D] = b_ref[...].T

    return pack


@functools.lru_cache(maxsize=None)
def _make_sc_scorer(total):
    assert total % (NW * 2 * CHUNK) == 0
    per_w = total // NW
    n_chunks = per_w // CHUNK
    mesh = plsc.VectorSubcoreMesh(core_axis_name="c", subcore_axis_name="s")

    rowbuf = pltpu.VMEM((CHUNK, 2 * D), jnp.float32)
    idxbuf = pltpu.VMEM((per_w,), jnp.int32)

    @functools.partial(
        pl.kernel,
        mesh=mesh,
        out_type=jax.ShapeDtypeStruct((total,), jnp.float32),
        compiler_params=pltpu.CompilerParams(needs_layout_passes=False),
        scratch_types=[
            idxbuf, idxbuf, idxbuf,
            idxbuf, idxbuf, idxbuf,
            [rowbuf] * 6,
            [rowbuf] * 6,
            pltpu.VMEM((per_w,), jnp.float32),
            pltpu.SemaphoreType.DMA,
            pltpu.SemaphoreType.DMA,
        ],
    )
    def scorer(hp_hbm, tp_hbm, rp_hbm, hc_hbm, tc_hbm, rc_hbm,
               eh_hbm, et_hbm, relh_hbm, relt_hbm,
               out_hbm, hpv, tpv, rpv, hcv, tcv, rcv,
               bufs0, bufs1, sv, sem0, sem1):
        wid = lax.axis_index("s") * NC + lax.axis_index("c")
        base = wid * per_w
        lanes = lax.iota(jnp.int32, L)
        bufsets = (bufs0, bufs1)
        sems = (sem0, sem1)

        pltpu.sync_copy(hp_hbm.at[pl.ds(base, per_w)], hpv)
        pltpu.sync_copy(tp_hbm.at[pl.ds(base, per_w)], tpv)
        pltpu.sync_copy(rp_hbm.at[pl.ds(base, per_w)], rpv)
        pltpu.sync_copy(hc_hbm.at[pl.ds(base, per_w)], hcv)
        pltpu.sync_copy(tc_hbm.at[pl.ds(base, per_w)], tcv)
        pltpu.sync_copy(rc_hbm.at[pl.ds(base, per_w)], rcv)

        def issue(c, which):
            off = pl.ds(c * CHUNK, CHUNK)
            bufs, sem = bufsets[which], sems[which]
            return [
                pltpu.async_copy(eh_hbm.at[hpv.at[off]], bufs[0], sem),
                pltpu.async_copy(et_hbm.at[hpv.at[off]], bufs[1], sem),
                pltpu.async_copy(eh_hbm.at[tpv.at[off]], bufs[2], sem),
                pltpu.async_copy(et_hbm.at[tpv.at[off]], bufs[3], sem),
                pltpu.async_copy(relh_hbm.at[rpv.at[off]], bufs[4], sem),
                pltpu.async_copy(relt_hbm.at[rpv.at[off]], bufs[5], sem),
            ]

        def compute(c, which):
            hh, ht, th, tt, rh, rt = bufsets[which]

            def group_body(g, carry):
                i0 = g * L
                s = pl.ds(c * CHUNK + i0, L)
                rows = lanes + i0
                hcol = hcv[s]
                tcol = tcv[s]
                rcol = rcv[s]
                acc = jnp.zeros((L,), jnp.float32)
                for d in range(D):
                    hhd = plsc.load_gather(hh, [rows, hcol + d])
                    htd = plsc.load_gather(ht, [rows, hcol + d])
                    thd = plsc.load_gather(th, [rows, tcol + d])
                    ttd = plsc.load_gather(tt, [rows, tcol + d])
                    rhd = plsc.load_gather(rh, [rows, rcol + d])
                    rtd = plsc.load_gather(rt, [rows, rcol + d])
                    acc = acc + (hhd * rhd * ttd + thd * rtd * htd)
                sv[pl.ds(c * CHUNK + i0, L)] = 0.5 * acc
                return carry

            lax.fori_loop(0, CHUNK // L, group_body, 0)

        def wait_chunk(c, which):
            off = pl.ds(c * CHUNK, CHUNK)
            bufs, sem = bufsets[which], sems[which]
            pltpu.make_async_copy(eh_hbm.at[hpv.at[off]], bufs[0], sem).wait()
            pltpu.make_async_copy(et_hbm.at[hpv.at[off]], bufs[1], sem).wait()
            pltpu.make_async_copy(eh_hbm.at[tpv.at[off]], bufs[2], sem).wait()
            pltpu.make_async_copy(et_hbm.at[tpv.at[off]], bufs[3], sem).wait()
            pltpu.make_async_copy(relh_hbm.at[rpv.at[off]], bufs[4],
                                  sem).wait()
            pltpu.make_async_copy(relt_hbm.at[rpv.at[off]], bufs[5],
                                  sem).wait()

        issue(0, 0)

        def pair(k, carry):
            c0 = 2 * k
            issue(c0 + 1, 1)
            wait_chunk(c0, 0)
            compute(c0, 0)
            nxt = jnp.minimum(c0 + 2, n_chunks - 2)
            issue(nxt, 0)
            wait_chunk(c0 + 1, 1)
            compute(c0 + 1, 1)
            return carry

        lax.fori_loop(0, n_chunks // 2, pair, 0)
        wait_chunk(n_chunks - 2, 0)
        pltpu.sync_copy(sv, out_hbm.at[pl.ds(base, per_w)])

    return scorer


def kernel(pos_h, pos_r, pos_t, neg_h, neg_r, neg_t,
           entity_head, entity_tail, relation_head, relation_tail):
    b = pos_h.shape[0]
    h = jnp.concatenate([pos_h, neg_h])
    t = jnp.concatenate([pos_t, neg_t])
    r = jnp.concatenate([pos_r, neg_r])
    sh = HALF.bit_length() - 1
    hp = ((h >> (sh + 1)) << sh) + (h & (HALF - 1))
    tp = ((t >> (sh + 1)) << sh) + (t & (HALF - 1))
    rp = ((r >> (sh + 1)) << sh) + (r & (HALF - 1))
    hc = ((h >> sh) & 1) * D
    tc = ((t >> sh) & 1) * D
    rc = ((r >> sh) & 1) * D
    epack = _make_tc_pack(entity_head.shape[0])
    rpack = _make_tc_pack(relation_head.shape[0])
    ehp = epack(entity_head.T, entity_head.T)
    etp = epack(entity_tail.T, entity_tail.T)
    rhp = rpack(relation_head.T, relation_head.T)
    rtp = rpack(relation_tail.T, relation_tail.T)
    scorer = _make_sc_scorer(2 * b)
    out = scorer(hp, tp, rp, hc, tc, rc, ehp, etp, rhp, rtp)
    return out[:b], out[b:]
